# Initial kernel scaffold; baseline (speedup 1.0000x reference)
import jax
import jax.numpy as jnp
from jax.experimental import pallas as pl


def kernel(z, embed_weight):
    raise NotImplementedError("write your pallas kernel here")


# speedup vs baseline: 1.9190x; 1.9190x over previous
import jax
import jax.numpy as jnp
from jax.experimental import pallas as pl

_GROUPS = 4
_BLK = 2048


def _vq_body(z_ref, e_ref, zq_ref, ind_ref, acc_ref):
    z = z_ref[...]
    e = e_ref[...]
    s = jax.lax.dot_general(z, e, (((1,), (1,)), ((), ())),
                            preferred_element_type=jnp.float32)
    zn = jnp.sum(z * z, axis=1, keepdims=True)
    en = jnp.sum(e * e, axis=1)
    d = (zn - 2.0 * s) + en[None, :]
    ind = jnp.argmax(-d, axis=1).astype(jnp.int32)
    onehot = (jax.lax.broadcasted_iota(jnp.int32, d.shape, 1)
              == ind[:, None]).astype(jnp.float32)
    zq = jax.lax.dot_general(onehot, e, (((1,), (0,)), ((), ())),
                             preferred_element_type=jnp.float32)
    zq_ref[...] = zq
    ind_ref[...] = ind
    r = zq - z
    part = jnp.sum(r * r).reshape(1, 1)

    @pl.when(pl.program_id(0) == 0)
    def _init():
        acc_ref[...] = part

    @pl.when(pl.program_id(0) != 0)
    def _accum():
        acc_ref[...] = acc_ref[...] + part


def kernel(z, embed_weight):
    b, n, d = z.shape
    cd = d // _GROUPS
    rows = b * n * _GROUPS
    ne = embed_weight.shape[0]
    flat = z.reshape(rows, cd)
    grid = rows // _BLK

    zq_flat, ind, acc = pl.pallas_call(
        _vq_body,
        grid=(grid,),
        in_specs=[
            pl.BlockSpec((_BLK, cd), lambda i: (i, 0)),
            pl.BlockSpec((ne, cd), lambda i: (0, 0)),
        ],
        out_specs=[
            pl.BlockSpec((_BLK, cd), lambda i: (i, 0)),
            pl.BlockSpec((_BLK,), lambda i: (i,)),
            pl.BlockSpec((1, 1), lambda i: (0, 0)),
        ],
        out_shape=[
            jax.ShapeDtypeStruct((rows, cd), jnp.float32),
            jax.ShapeDtypeStruct((rows,), jnp.int32),
            jax.ShapeDtypeStruct((1, 1), jnp.float32),
        ],
    )(flat, embed_weight)

    mse = acc[0, 0] / (rows * cd)
    diff = (0.25 * mse + mse) * 10.0
    zq = zq_flat.reshape(b, n, d)
    ind_out = ind.reshape(n, b, _GROUPS)
    return zq, diff, ind_out


# speedup vs baseline: 2.0656x; 1.0764x over previous
import functools

import jax
import jax.numpy as jnp
from jax import lax
from jax.experimental import pallas as pl
from jax.experimental.pallas import tpu as pltpu
from jax.experimental.pallas import tpu_sc as plsc

_GROUPS = 4
_BLK = 2048
_NC = 2
_NS = 16


def _vq_dist_body(z_ref, e_ref, ind_ref, acc_ref):
    z = z_ref[...]
    e = e_ref[...]
    s = lax.dot_general(z, e, (((1,), (1,)), ((), ())),
                        preferred_element_type=jnp.float32)
    zn = jnp.sum(z * z, axis=1, keepdims=True)
    en = jnp.sum(e * e, axis=1)
    d = (zn - 2.0 * s) + en[None, :]
    neg = -d
    ind_ref[...] = jnp.argmax(neg, axis=1).astype(jnp.int32)
    part = (-jnp.sum(jnp.max(neg, axis=1))).reshape(1, 1)

    @pl.when(pl.program_id(0) == 0)
    def _init():
        acc_ref[...] = part

    @pl.when(pl.program_id(0) != 0)
    def _accum():
        acc_ref[...] = acc_ref[...] + part


def _make_sc_gather(rows, cd, chunks, chunk):
    mesh = plsc.VectorSubcoreMesh(core_axis_name="c", subcore_axis_name="s",
                                  num_cores=_NC, num_subcores=_NS)

    @functools.partial(
        pl.kernel,
        out_type=jax.ShapeDtypeStruct((rows, cd), jnp.float32),
        mesh=mesh,
        scratch_types=[
            pltpu.VMEM((chunk,), jnp.int32),
            pltpu.VMEM((chunk, cd), jnp.float32),
            pltpu.SemaphoreType.DMA,
        ],
        compiler_params=pltpu.CompilerParams(use_tc_tiling_on_sc=False),
    )
    def sc_gather(table_hbm, idx_hbm, out_hbm, idx_v, rows_v, sem):
        wid = lax.axis_index("s") * _NC + lax.axis_index("c")
        for c in range(chunks):
            base = (wid * chunks + c) * chunk
            pltpu.sync_copy(idx_hbm.at[pl.ds(base, chunk)], idx_v)
            pltpu.async_copy(table_hbm.at[idx_v], rows_v, sem).wait()
            pltpu.sync_copy(rows_v, out_hbm.at[pl.ds(base, chunk)])

    return sc_gather


def kernel(z, embed_weight):
    b, n, d = z.shape
    cd = d // _GROUPS
    rows = b * n * _GROUPS
    ne = embed_weight.shape[0]
    flat = z.reshape(rows, cd)
    grid = rows // _BLK

    ind, acc = pl.pallas_call(
        _vq_dist_body,
        grid=(grid,),
        in_specs=[
            pl.BlockSpec((_BLK, cd), lambda i: (i, 0)),
            pl.BlockSpec((ne, cd), lambda i: (0, 0)),
        ],
        out_specs=[
            pl.BlockSpec((_BLK,), lambda i: (i,)),
            pl.BlockSpec((1, 1), lambda i: (0, 0)),
        ],
        out_shape=[
            jax.ShapeDtypeStruct((rows,), jnp.int32),
            jax.ShapeDtypeStruct((1, 1), jnp.float32),
        ],
    )(flat, embed_weight)

    n_workers = _NC * _NS
    per_w = rows // n_workers
    chunk = 1024
    chunks = per_w // chunk
    zq_flat = _make_sc_gather(rows, cd, chunks, chunk)(embed_weight, ind)

    mse = acc[0, 0] / (rows * cd)
    diff = (0.25 * mse + mse) * 10.0
    zq = zq_flat.reshape(b, n, d)
    ind_out = ind.reshape(n, b, _GROUPS)
    return zq, diff, ind_out


# speedup vs baseline: 2.2405x; 1.0847x over previous
import functools

import jax
import jax.numpy as jnp
from jax import lax
from jax.experimental import pallas as pl
from jax.experimental.pallas import tpu as pltpu
from jax.experimental.pallas import tpu_sc as plsc

_GROUPS = 4
_BLK = 2048
_NC = 2
_NS = 16


def _vq_dist_body(z_ref, e_ref, ind_ref, acc_ref):
    z = z_ref[...]
    e = e_ref[...]
    zn = jnp.sum(z * z, axis=1, keepdims=True)
    en = jnp.sum(e * e, axis=1)
    s = lax.dot_general(z, e, (((1,), (1,)), ((), ())),
                        preferred_element_type=jnp.float32)
    d = (zn - 2.0 * s) + en[None, :]
    md = jnp.min(d, axis=1)
    lane = jax.lax.broadcasted_iota(jnp.int32, d.shape, 1)
    ind_ref[...] = jnp.min(jnp.where(d == md[:, None], lane, d.shape[1]),
                           axis=1)
    part = jnp.sum(md).reshape(1, 1)

    @pl.when(pl.program_id(0) == 0)
    def _init():
        acc_ref[...] = part

    @pl.when(pl.program_id(0) != 0)
    def _accum():
        acc_ref[...] = acc_ref[...] + part


def _make_sc_gather(rows, cd, chunks, chunk):
    mesh = plsc.VectorSubcoreMesh(core_axis_name="c", subcore_axis_name="s",
                                  num_cores=_NC, num_subcores=_NS)

    @functools.partial(
        pl.kernel,
        out_type=jax.ShapeDtypeStruct((rows, cd), jnp.float32),
        mesh=mesh,
        scratch_types=[
            pltpu.VMEM((chunk,), jnp.int32),
            pltpu.VMEM((chunk, cd), jnp.float32),
            pltpu.SemaphoreType.DMA,
        ],
        compiler_params=pltpu.CompilerParams(use_tc_tiling_on_sc=False),
    )
    def sc_gather(table_hbm, idx_hbm, out_hbm, idx_v, rows_v, sem):
        wid = lax.axis_index("s") * _NC + lax.axis_index("c")
        for c in range(chunks):
            base = (wid * chunks + c) * chunk
            pltpu.sync_copy(idx_hbm.at[pl.ds(base, chunk)], idx_v)
            pltpu.async_copy(table_hbm.at[idx_v], rows_v, sem).wait()
            pltpu.sync_copy(rows_v, out_hbm.at[pl.ds(base, chunk)])

    return sc_gather


def kernel(z, embed_weight):
    b, n, d = z.shape
    cd = d // _GROUPS
    rows = b * n * _GROUPS
    ne = embed_weight.shape[0]
    flat = z.reshape(rows, cd)
    grid = rows // _BLK

    ind, acc = pl.pallas_call(
        _vq_dist_body,
        grid=(grid,),
        in_specs=[
            pl.BlockSpec((_BLK, cd), lambda i: (i, 0)),
            pl.BlockSpec((ne, cd), lambda i: (0, 0)),
        ],
        out_specs=[
            pl.BlockSpec((_BLK,), lambda i: (i,)),
            pl.BlockSpec((1, 1), lambda i: (0, 0)),
        ],
        out_shape=[
            jax.ShapeDtypeStruct((rows,), jnp.int32),
            jax.ShapeDtypeStruct((1, 1), jnp.float32),
        ],
    )(flat, embed_weight)

    n_workers = _NC * _NS
    per_w = rows // n_workers
    chunk = 1024
    chunks = per_w // chunk
    zq_flat = _make_sc_gather(rows, cd, chunks, chunk)(embed_weight, ind)

    mse = acc[0, 0] / (rows * cd)
    diff = (0.25 * mse + mse) * 10.0
    zq = zq_flat.reshape(b, n, d)
    ind_out = ind.reshape(n, b, _GROUPS)
    return zq, diff, ind_out


# speedup vs baseline: 2.5701x; 1.1471x over previous
import functools

import jax
import jax.numpy as jnp
from jax import lax
from jax.experimental import pallas as pl
from jax.experimental.pallas import tpu as pltpu
from jax.experimental.pallas import tpu_sc as plsc

_GROUPS = 4
_BLK = 512
_NC = 2
_NS = 16


def _vq_dist_body(z_ref, e_ref, i0_ref, i1_ref, i2_ref, i3_ref, acc_ref):
    zb = z_ref[...]
    e = e_ref[...]
    cd = e.shape[1]
    ne = e.shape[0]
    en = jnp.sum(e * e, axis=1)
    ind_refs = (i0_ref, i1_ref, i2_ref, i3_ref)
    part = jnp.zeros((1, 1), jnp.float32)
    for g in range(_GROUPS):
        zg = zb[:, g * cd:(g + 1) * cd]
        s = lax.dot_general(zg, e, (((1,), (1,)), ((), ())),
                            preferred_element_type=jnp.float32)
        zn = jnp.sum(zg * zg, axis=1, keepdims=True)
        d = (zn - 2.0 * s) + en[None, :]
        md = jnp.min(d, axis=1)
        lane = lax.broadcasted_iota(jnp.int32, d.shape, 1)
        ind_refs[g][...] = jnp.min(jnp.where(d == md[:, None], lane, ne),
                                   axis=1)
        part = part + jnp.sum(md).reshape(1, 1)

    @pl.when(pl.program_id(0) == 0)
    def _init():
        acc_ref[...] = part

    @pl.when(pl.program_id(0) != 0)
    def _accum():
        acc_ref[...] = acc_ref[...] + part


def _make_sc_gather(q_rows, d_full, cd, per_w):
    mesh = plsc.VectorSubcoreMesh(core_axis_name="c", subcore_axis_name="s",
                                  num_cores=_NC, num_subcores=_NS)

    @functools.partial(
        pl.kernel,
        out_type=jax.ShapeDtypeStruct((q_rows, d_full), jnp.float32),
        mesh=mesh,
        scratch_types=[
            pltpu.VMEM((per_w,), jnp.int32),
            pltpu.VMEM((per_w, cd), jnp.float32),
            pltpu.SemaphoreType.DMA,
        ],
        compiler_params=pltpu.CompilerParams(use_tc_tiling_on_sc=False),
    )
    def sc_gather(table_hbm, i0_hbm, i1_hbm, i2_hbm, i3_hbm, out_hbm,
                  idx_v, rows_v, sem):
        wid = lax.axis_index("s") * _NC + lax.axis_index("c")
        base = wid * per_w
        idx_hbms = (i0_hbm, i1_hbm, i2_hbm, i3_hbm)
        for g in range(_GROUPS):
            pltpu.sync_copy(idx_hbms[g].at[pl.ds(base, per_w)], idx_v)
            pltpu.async_copy(table_hbm.at[idx_v], rows_v, sem).wait()
            pltpu.sync_copy(rows_v,
                            out_hbm.at[pl.ds(base, per_w),
                                       pl.ds(g * cd, cd)])

    return sc_gather


def kernel(z, embed_weight):
    b, n, d_full = z.shape
    cd = d_full // _GROUPS
    q_rows = b * n
    ne = embed_weight.shape[0]
    z2 = z.reshape(q_rows, d_full)
    grid = q_rows // _BLK

    ind_shape = jax.ShapeDtypeStruct((q_rows,), jnp.int32)
    ind_spec = pl.BlockSpec((_BLK,), lambda i: (i,))
    i0, i1, i2, i3, acc = pl.pallas_call(
        _vq_dist_body,
        grid=(grid,),
        in_specs=[
            pl.BlockSpec((_BLK, d_full), lambda i: (i, 0)),
            pl.BlockSpec((ne, cd), lambda i: (0, 0)),
        ],
        out_specs=[ind_spec, ind_spec, ind_spec, ind_spec,
                   pl.BlockSpec((1, 1), lambda i: (0, 0))],
        out_shape=[ind_shape, ind_shape, ind_shape, ind_shape,
                   jax.ShapeDtypeStruct((1, 1), jnp.float32)],
    )(z2, embed_weight)

    per_w = q_rows // (_NC * _NS)
    zq2 = _make_sc_gather(q_rows, d_full, cd, per_w)(
        embed_weight, i0, i1, i2, i3)

    mse = acc[0, 0] / (q_rows * d_full)
    diff = (0.25 * mse + mse) * 10.0
    zq = zq2.reshape(b, n, d_full)
    ind_out = jnp.stack([i0, i1, i2, i3], axis=1).reshape(n, b, _GROUPS)
    return zq, diff, ind_out


# speedup vs baseline: 2.8350x; 1.1031x over previous
import functools

import jax
import jax.numpy as jnp
from jax import lax
from jax.experimental import pallas as pl
from jax.experimental.pallas import tpu as pltpu
from jax.experimental.pallas import tpu_sc as plsc

_GROUPS = 4
_BLK = 1024
_NC = 2
_NS = 16


def _vq_dist_body(z_ref, e_ref, i0_ref, i1_ref, i2_ref, i3_ref, acc_ref):
    zb = z_ref[...]
    e = e_ref[...]
    cd = e.shape[1]
    ne = e.shape[0]
    en = jnp.sum(e * e, axis=1)
    e2 = e + e
    ind_refs = (i0_ref, i1_ref, i2_ref, i3_ref)
    part = jnp.zeros((1, 1), jnp.float32)
    lane = lax.broadcasted_iota(jnp.int32, (zb.shape[0], ne), 1)
    for g in range(_GROUPS):
        zg = zb[:, g * cd:(g + 1) * cd]
        s2 = lax.dot_general(zg, e2, (((1,), (1,)), ((), ())),
                             preferred_element_type=jnp.float32)
        zn = jnp.sum(zg * zg, axis=1, keepdims=True)
        d = (zn - s2) + en[None, :]
        md = jnp.min(d, axis=1)
        ind_refs[g][...] = jnp.min(jnp.where(d == md[:, None], lane, ne),
                                   axis=1)
        part = part + jnp.sum(md).reshape(1, 1)

    @pl.when(pl.program_id(0) == 0)
    def _init():
        acc_ref[...] = part

    @pl.when(pl.program_id(0) != 0)
    def _accum():
        acc_ref[...] = acc_ref[...] + part


def _make_sc_gather(q_rows, d_full, cd, per_w):
    mesh = plsc.VectorSubcoreMesh(core_axis_name="c", subcore_axis_name="s",
                                  num_cores=_NC, num_subcores=_NS)

    @functools.partial(
        pl.kernel,
        out_type=jax.ShapeDtypeStruct((q_rows, d_full), jnp.float32),
        mesh=mesh,
        scratch_types=[
            pltpu.VMEM((per_w,), jnp.int32),
            pltpu.VMEM((per_w, cd), jnp.float32),
            pltpu.SemaphoreType.DMA,
        ],
        compiler_params=pltpu.CompilerParams(use_tc_tiling_on_sc=False),
    )
    def sc_gather(table_hbm, i0_hbm, i1_hbm, i2_hbm, i3_hbm, out_hbm,
                  idx_v, rows_v, sem):
        wid = lax.axis_index("s") * _NC + lax.axis_index("c")
        base = wid * per_w
        idx_hbms = (i0_hbm, i1_hbm, i2_hbm, i3_hbm)
        for g in range(_GROUPS):
            pltpu.sync_copy(idx_hbms[g].at[pl.ds(base, per_w)], idx_v)
            pltpu.async_copy(table_hbm.at[idx_v], rows_v, sem).wait()
            pltpu.sync_copy(rows_v,
                            out_hbm.at[pl.ds(base, per_w),
                                       pl.ds(g * cd, cd)])

    return sc_gather


def kernel(z, embed_weight):
    b, n, d_full = z.shape
    cd = d_full // _GROUPS
    q_rows = b * n
    ne = embed_weight.shape[0]
    z2 = z.reshape(q_rows, d_full)
    grid = q_rows // _BLK

    ind_shape = jax.ShapeDtypeStruct((q_rows,), jnp.int32)
    ind_spec = pl.BlockSpec((_BLK,), lambda i: (i,))
    i0, i1, i2, i3, acc = pl.pallas_call(
        _vq_dist_body,
        grid=(grid,),
        in_specs=[
            pl.BlockSpec((_BLK, d_full), lambda i: (i, 0)),
            pl.BlockSpec((ne, cd), lambda i: (0, 0)),
        ],
        out_specs=[ind_spec, ind_spec, ind_spec, ind_spec,
                   pl.BlockSpec((1, 1), lambda i: (0, 0))],
        out_shape=[ind_shape, ind_shape, ind_shape, ind_shape,
                   jax.ShapeDtypeStruct((1, 1), jnp.float32)],
    )(z2, embed_weight)

    per_w = q_rows // (_NC * _NS)
    zq2 = _make_sc_gather(q_rows, d_full, cd, per_w)(
        embed_weight, i0, i1, i2, i3)

    mse = acc[0, 0] / (q_rows * d_full)
    diff = (0.25 * mse + mse) * 10.0
    zq = zq2.reshape(b, n, d_full)
    ind_out = jnp.stack([i0, i1, i2, i3], axis=1).reshape(n, b, _GROUPS)
    return zq, diff, ind_out


# speedup vs baseline: 2.8539x; 1.0067x over previous
import functools

import jax
import jax.numpy as jnp
from jax import lax
from jax.experimental import pallas as pl
from jax.experimental.pallas import tpu as pltpu
from jax.experimental.pallas import tpu_sc as plsc

_GROUPS = 4
_BLK = 2048
_NC = 2
_NS = 16


def _vq_dist_body(z_ref, e_ref, i0_ref, i1_ref, i2_ref, i3_ref, acc_ref):
    zb = z_ref[...]
    e = e_ref[...]
    cd = e.shape[1]
    ne = e.shape[0]
    en = jnp.sum(e * e, axis=1)
    e2 = e + e
    ind_refs = (i0_ref, i1_ref, i2_ref, i3_ref)
    part = jnp.zeros((1, 1), jnp.float32)
    lane = lax.broadcasted_iota(jnp.int32, (zb.shape[0], ne), 1)
    for g in range(_GROUPS):
        zg = zb[:, g * cd:(g + 1) * cd]
        s2 = lax.dot_general(zg, e2, (((1,), (1,)), ((), ())),
                             preferred_element_type=jnp.float32)
        zn = jnp.sum(zg * zg, axis=1, keepdims=True)
        d = (zn - s2) + en[None, :]
        md = jnp.min(d, axis=1)
        ind_refs[g][...] = jnp.min(jnp.where(d == md[:, None], lane, ne),
                                   axis=1)
        part = part + jnp.sum(md).reshape(1, 1)

    @pl.when(pl.program_id(0) == 0)
    def _init():
        acc_ref[...] = part

    @pl.when(pl.program_id(0) != 0)
    def _accum():
        acc_ref[...] = acc_ref[...] + part


def _make_sc_gather(q_rows, d_full, cd, per_w):
    mesh = plsc.VectorSubcoreMesh(core_axis_name="c", subcore_axis_name="s",
                                  num_cores=_NC, num_subcores=_NS)

    @functools.partial(
        pl.kernel,
        out_type=jax.ShapeDtypeStruct((q_rows, d_full), jnp.float32),
        mesh=mesh,
        scratch_types=[
            pltpu.VMEM((per_w,), jnp.int32),
            pltpu.VMEM((per_w,), jnp.int32),
            pltpu.VMEM((per_w, cd), jnp.float32),
            pltpu.VMEM((per_w, cd), jnp.float32),
            pltpu.SemaphoreType.DMA,
            pltpu.SemaphoreType.DMA,
            pltpu.SemaphoreType.DMA,
            pltpu.SemaphoreType.DMA,
        ],
        compiler_params=pltpu.CompilerParams(use_tc_tiling_on_sc=False),
    )
    def sc_gather(table_hbm, i0_hbm, i1_hbm, i2_hbm, i3_hbm, out_hbm,
                  idx_a, idx_b, rows_a, rows_b, ga, gb, wa, wb):
        wid = lax.axis_index("s") * _NC + lax.axis_index("c")
        base = wid * per_w
        idx_hbms = (i0_hbm, i1_hbm, i2_hbm, i3_hbm)
        idxs = (idx_a, idx_b)
        rows = (rows_a, rows_b)
        gsems = (ga, gb)
        wsems = (wa, wb)

        def out_slice(g):
            return out_hbm.at[pl.ds(base, per_w), pl.ds(g * cd, cd)]

        gh = [None, None]
        wh = [None, None]
        for g in range(_GROUPS):
            buf = g & 1
            if wh[buf] is not None:
                wh[buf].wait()
            pltpu.sync_copy(idx_hbms[g].at[pl.ds(base, per_w)], idxs[buf])
            gh[buf] = pltpu.async_copy(table_hbm.at[idxs[buf]], rows[buf],
                                       gsems[buf])
            if g >= 1:
                prev = 1 - buf
                gh[prev].wait()
                wh[prev] = pltpu.async_copy(rows[prev], out_slice(g - 1),
                                            wsems[prev])
        last = (_GROUPS - 1) & 1
        gh[last].wait()
        wh[last] = pltpu.async_copy(rows[last], out_slice(_GROUPS - 1),
                                    wsems[last])
        wh[0].wait()
        wh[1].wait()

    return sc_gather


def kernel(z, embed_weight):
    b, n, d_full = z.shape
    cd = d_full // _GROUPS
    q_rows = b * n
    ne = embed_weight.shape[0]
    z2 = z.reshape(q_rows, d_full)
    grid = q_rows // _BLK

    ind_shape = jax.ShapeDtypeStruct((q_rows,), jnp.int32)
    ind_spec = pl.BlockSpec((_BLK,), lambda i: (i,))
    i0, i1, i2, i3, acc = pl.pallas_call(
        _vq_dist_body,
        grid=(grid,),
        in_specs=[
            pl.BlockSpec((_BLK, d_full), lambda i: (i, 0)),
            pl.BlockSpec((ne, cd), lambda i: (0, 0)),
        ],
        out_specs=[ind_spec, ind_spec, ind_spec, ind_spec,
                   pl.BlockSpec((1, 1), lambda i: (0, 0))],
        out_shape=[ind_shape, ind_shape, ind_shape, ind_shape,
                   jax.ShapeDtypeStruct((1, 1), jnp.float32)],
    )(z2, embed_weight)

    per_w = q_rows // (_NC * _NS)
    zq2 = _make_sc_gather(q_rows, d_full, cd, per_w)(
        embed_weight, i0, i1, i2, i3)

    mse = acc[0, 0] / (q_rows * d_full)
    diff = (0.25 * mse + mse) * 10.0
    zq = zq2.reshape(b, n, d_full)
    ind_out = jnp.stack([i0, i1, i2, i3], axis=1).reshape(n, b, _GROUPS)
    return zq, diff, ind_out
